# baseline (speedup 1.0000x reference)
import jax
import jax.numpy as jnp
from jax.experimental import pallas as pl
from jax.experimental.pallas import tpu as pltpu

_DELTA = 0.1
_EPS = 1e-12
_BR = 128
_BC = 2048


def _ghm_body(x_rows_ref, y_rows_ref, x_cols_ref, y_cols_ref, pw_ref,
              wsum_ref, psum_ref, gcols_ref):
    i = pl.program_id(0)
    n = x_cols_ref.shape[1]

    @pl.when(i == 0)
    def _init():
        xc = x_cols_ref[...]
        yc = y_cols_ref[...]
        gcols_ref[...] = jnp.abs(jax.nn.sigmoid(xc) - yc)
        wsum_ref[0, 0] = 0.0
        psum_ref[0, 0] = 0.0

    x_r = x_rows_ref[0]
    y_r = y_rows_ref[0]
    g_r = jnp.abs(jax.nn.sigmoid(x_r) - y_r)

    def col_step(c, acc):
        off = pl.multiple_of(c * _BC, _BC)
        gc = gcols_ref[:, pl.ds(off, _BC)]
        m = (jnp.abs(g_r - gc) <= _DELTA).astype(jnp.float32)
        t = m[:, 0:128]
        for s in range(1, _BC // 128):
            t = t + m[:, s * 128:(s + 1) * 128]
        return acc + t

    cnt128 = jax.lax.fori_loop(0, n // _BC, col_step,
                               jnp.zeros((_BR, 128), jnp.float32))
    cnt = jnp.sum(cnt128, axis=1, keepdims=True)

    gd = cnt / _DELTA
    beta = n / (gd + _EPS)

    pw = pw_ref[0, 0]
    pe = pw * y_r * jax.nn.softplus(-x_r) + (1.0 - y_r) * jax.nn.softplus(x_r)

    wsum_ref[0, 0] += jnp.sum(beta * pe)
    psum_ref[0, 0] += jnp.sum(pe)


def kernel(logits, targets, pos_weight):
    x = logits.reshape(-1).astype(jnp.float32)
    y = targets.reshape(-1).astype(jnp.float32)
    n = x.shape[0]
    g = n // _BR

    x_rows = x.reshape(g, _BR, 1)
    y_rows = y.reshape(g, _BR, 1)
    x_cols = x.reshape(1, n)
    y_cols = y.reshape(1, n)
    pw = jnp.asarray(pos_weight, jnp.float32).reshape(1, 1)

    wsum, psum = pl.pallas_call(
        _ghm_body,
        grid=(g,),
        in_specs=[
            pl.BlockSpec((1, _BR, 1), lambda i: (i, 0, 0)),
            pl.BlockSpec((1, _BR, 1), lambda i: (i, 0, 0)),
            pl.BlockSpec((1, n), lambda i: (0, 0)),
            pl.BlockSpec((1, n), lambda i: (0, 0)),
            pl.BlockSpec(memory_space=pltpu.SMEM),
        ],
        out_specs=[
            pl.BlockSpec((1, 1), lambda i: (0, 0), memory_space=pltpu.SMEM),
            pl.BlockSpec((1, 1), lambda i: (0, 0), memory_space=pltpu.SMEM),
        ],
        out_shape=[
            jax.ShapeDtypeStruct((1, 1), jnp.float32),
            jax.ShapeDtypeStruct((1, 1), jnp.float32),
        ],
        scratch_shapes=[pltpu.VMEM((1, n), jnp.float32)],
        compiler_params=pltpu.CompilerParams(
            dimension_semantics=("arbitrary",),
        ),
        name="ghm_bce",
    )(x_rows, y_rows, x_cols, y_cols, pw)

    inv_n = jnp.float32(1.0 / n)
    return wsum[0, 0] * inv_n, psum[0, 0] * inv_n


# speedup vs baseline: 1.2310x; 1.2310x over previous
import jax
import jax.numpy as jnp
from jax.experimental import pallas as pl
from jax.experimental.pallas import tpu as pltpu

_DELTA = 0.1
_EPS = 1e-12
_BR = 256
_BC = 2048


def _slab_sum(m):
    t = m[:, 0:128]
    for s in range(1, m.shape[1] // 128):
        t = t + m[:, s * 128:(s + 1) * 128]
    return t


def _ghm_body(x_rows_ref, y_rows_ref, x_cols_ref, y_cols_ref, pw_ref,
              wsum_ref, psum_ref, gcols_ref, colcnt_ref, rowcnt_ref):
    i = pl.program_id(0)
    n = x_cols_ref.shape[1]
    ng = pl.num_programs(0)

    @pl.when(i == 0)
    def _init():
        xc = x_cols_ref[...]
        yc = y_cols_ref[...]
        gcols_ref[...] = jnp.abs(jax.nn.sigmoid(xc) - yc)
        colcnt_ref[...] = jnp.zeros_like(colcnt_ref)

    x_r = x_rows_ref[0]
    y_r = y_rows_ref[0]
    g_r = jnp.abs(jax.nn.sigmoid(x_r) - y_r)

    r0 = i * _BR
    c0 = r0 // _BC

    off0 = pl.multiple_of(c0 * _BC, _BC)
    gc0 = gcols_ref[:, pl.ds(off0, _BC)]
    m0 = (jnp.abs(g_r - gc0) <= _DELTA).astype(jnp.float32)
    lane = jax.lax.broadcasted_iota(jnp.int32, (1, _BC), 1) + off0
    row_keep = (lane >= r0).astype(jnp.float32)
    col_keep = (lane >= r0 + _BR).astype(jnp.float32)
    acc = _slab_sum(m0 * row_keep)
    colcnt_ref[:, pl.ds(off0, _BC)] += jnp.sum(
        m0 * col_keep, axis=0, keepdims=True)

    def chunk(c, a):
        off = pl.multiple_of(c * _BC, _BC)
        gc = gcols_ref[:, pl.ds(off, _BC)]
        m = (jnp.abs(g_r - gc) <= _DELTA).astype(jnp.float32)
        colcnt_ref[:, pl.ds(off, _BC)] += jnp.sum(m, axis=0, keepdims=True)
        return a + _slab_sum(m)

    acc = jax.lax.fori_loop(c0 + 1, n // _BC, chunk, acc)

    cnt = jnp.sum(acc, axis=1, keepdims=True)
    rowcnt_ref[:, pl.ds(r0, _BR)] = cnt.reshape(1, _BR)

    @pl.when(i == ng - 1)
    def _finalize():
        gd = (rowcnt_ref[...] + colcnt_ref[...]) / _DELTA
        beta = n / (gd + _EPS)
        xc = x_cols_ref[...]
        yc = y_cols_ref[...]
        pw = pw_ref[0, 0]
        pe = (pw * yc * jax.nn.softplus(-xc)
              + (1.0 - yc) * jax.nn.softplus(xc))
        wsum_ref[0, 0] = jnp.sum(beta * pe)
        psum_ref[0, 0] = jnp.sum(pe)


def kernel(logits, targets, pos_weight):
    x = logits.reshape(-1).astype(jnp.float32)
    y = targets.reshape(-1).astype(jnp.float32)
    n = x.shape[0]
    g = n // _BR

    x_rows = x.reshape(g, _BR, 1)
    y_rows = y.reshape(g, _BR, 1)
    x_cols = x.reshape(1, n)
    y_cols = y.reshape(1, n)
    pw = jnp.asarray(pos_weight, jnp.float32).reshape(1, 1)

    wsum, psum = pl.pallas_call(
        _ghm_body,
        grid=(g,),
        in_specs=[
            pl.BlockSpec((1, _BR, 1), lambda i: (i, 0, 0)),
            pl.BlockSpec((1, _BR, 1), lambda i: (i, 0, 0)),
            pl.BlockSpec((1, n), lambda i: (0, 0)),
            pl.BlockSpec((1, n), lambda i: (0, 0)),
            pl.BlockSpec(memory_space=pltpu.SMEM),
        ],
        out_specs=[
            pl.BlockSpec((1, 1), lambda i: (0, 0), memory_space=pltpu.SMEM),
            pl.BlockSpec((1, 1), lambda i: (0, 0), memory_space=pltpu.SMEM),
        ],
        out_shape=[
            jax.ShapeDtypeStruct((1, 1), jnp.float32),
            jax.ShapeDtypeStruct((1, 1), jnp.float32),
        ],
        scratch_shapes=[
            pltpu.VMEM((1, n), jnp.float32),
            pltpu.VMEM((1, n), jnp.float32),
            pltpu.VMEM((1, n), jnp.float32),
        ],
        compiler_params=pltpu.CompilerParams(
            dimension_semantics=("arbitrary",),
        ),
        name="ghm_bce",
    )(x_rows, y_rows, x_cols, y_cols, pw)

    inv_n = jnp.float32(1.0 / n)
    return wsum[0, 0] * inv_n, psum[0, 0] * inv_n


# speedup vs baseline: 1.4140x; 1.1487x over previous
import jax
import jax.numpy as jnp
from jax.experimental import pallas as pl
from jax.experimental.pallas import tpu as pltpu

_DELTA = 0.1
_EPS = 1e-12
_BR = 128
_BC = 2048


def _count_chunk(gcols_ref, g_r, off, acc, lane_keep=None):
    colparts = []
    for s in range(_BC // 128):
        gc = gcols_ref[:, pl.ds(off + s * 128, 128)]
        m = (jnp.abs(g_r - gc) <= _DELTA).astype(jnp.float32)
        if lane_keep is None:
            acc = acc + m
            colparts.append(jnp.sum(m, axis=0, keepdims=True))
        else:
            row_keep, col_keep = lane_keep
            acc = acc + m * row_keep[:, s * 128:(s + 1) * 128]
            colparts.append(jnp.sum(m * col_keep[:, s * 128:(s + 1) * 128],
                                    axis=0, keepdims=True))
    return acc, jnp.concatenate(colparts, axis=1)


def _ghm_body(x_rows_ref, y_rows_ref, x_cols_ref, y_cols_ref, pw_ref,
              wsum_ref, psum_ref, gcols_ref, colcnt_ref, rowcnt_ref):
    i = pl.program_id(0)
    n = x_cols_ref.shape[1]
    ng = pl.num_programs(0)

    @pl.when(i == 0)
    def _init():
        xc = x_cols_ref[...]
        yc = y_cols_ref[...]
        gcols_ref[...] = jnp.abs(jax.nn.sigmoid(xc) - yc)
        colcnt_ref[...] = jnp.zeros_like(colcnt_ref)

    x_r = x_rows_ref[0]
    y_r = y_rows_ref[0]
    g_r = jnp.abs(jax.nn.sigmoid(x_r) - y_r)

    r0 = i * _BR
    c0 = r0 // _BC

    off0 = pl.multiple_of(c0 * _BC, _BC)
    lane = jax.lax.broadcasted_iota(jnp.int32, (1, _BC), 1) + off0
    row_keep = (lane >= r0).astype(jnp.float32)
    col_keep = (lane >= r0 + _BR).astype(jnp.float32)
    acc, colsum0 = _count_chunk(gcols_ref, g_r, off0,
                                jnp.zeros((_BR, 128), jnp.float32),
                                (row_keep, col_keep))
    colcnt_ref[:, pl.ds(off0, _BC)] += colsum0

    def chunk(c, a):
        off = pl.multiple_of(c * _BC, _BC)
        a, colsum = _count_chunk(gcols_ref, g_r, off, a)
        colcnt_ref[:, pl.ds(off, _BC)] += colsum
        return a

    acc = jax.lax.fori_loop(c0 + 1, n // _BC, chunk, acc)

    cnt = jnp.sum(acc, axis=1, keepdims=True)
    rowcnt_ref[:, pl.ds(r0, _BR)] = cnt.reshape(1, _BR)

    @pl.when(i == ng - 1)
    def _finalize():
        gd = (rowcnt_ref[...] + colcnt_ref[...]) / _DELTA
        beta = n / (gd + _EPS)
        xc = x_cols_ref[...]
        yc = y_cols_ref[...]
        pw = pw_ref[0, 0]
        pe = (pw * yc * jax.nn.softplus(-xc)
              + (1.0 - yc) * jax.nn.softplus(xc))
        wsum_ref[0, 0] = jnp.sum(beta * pe)
        psum_ref[0, 0] = jnp.sum(pe)


def kernel(logits, targets, pos_weight):
    x = logits.reshape(-1).astype(jnp.float32)
    y = targets.reshape(-1).astype(jnp.float32)
    n = x.shape[0]
    g = n // _BR

    x_rows = x.reshape(g, _BR, 1)
    y_rows = y.reshape(g, _BR, 1)
    x_cols = x.reshape(1, n)
    y_cols = y.reshape(1, n)
    pw = jnp.asarray(pos_weight, jnp.float32).reshape(1, 1)

    wsum, psum = pl.pallas_call(
        _ghm_body,
        grid=(g,),
        in_specs=[
            pl.BlockSpec((1, _BR, 1), lambda i: (i, 0, 0)),
            pl.BlockSpec((1, _BR, 1), lambda i: (i, 0, 0)),
            pl.BlockSpec((1, n), lambda i: (0, 0)),
            pl.BlockSpec((1, n), lambda i: (0, 0)),
            pl.BlockSpec(memory_space=pltpu.SMEM),
        ],
        out_specs=[
            pl.BlockSpec((1, 1), lambda i: (0, 0), memory_space=pltpu.SMEM),
            pl.BlockSpec((1, 1), lambda i: (0, 0), memory_space=pltpu.SMEM),
        ],
        out_shape=[
            jax.ShapeDtypeStruct((1, 1), jnp.float32),
            jax.ShapeDtypeStruct((1, 1), jnp.float32),
        ],
        scratch_shapes=[
            pltpu.VMEM((1, n), jnp.float32),
            pltpu.VMEM((1, n), jnp.float32),
            pltpu.VMEM((1, n), jnp.float32),
        ],
        compiler_params=pltpu.CompilerParams(
            dimension_semantics=("arbitrary",),
        ),
        name="ghm_bce",
    )(x_rows, y_rows, x_cols, y_cols, pw)

    inv_n = jnp.float32(1.0 / n)
    return wsum[0, 0] * inv_n, psum[0, 0] * inv_n


# speedup vs baseline: 1.5172x; 1.0730x over previous
import jax
import jax.numpy as jnp
from jax.experimental import pallas as pl
from jax.experimental.pallas import tpu as pltpu

_DELTA = 0.1
_EPS = 1e-12
_BR = 512
_BC = 2048


def _count_chunk(gcols_ref, colcnt_ref, g_r, off, acc, d=None):
    for s in range(_BC // 128):
        gc = gcols_ref[:, pl.ds(off + s * 128, 128)]
        m = (jnp.abs(g_r - gc) <= _DELTA).astype(jnp.float32)
        if d is None:
            acc = acc + m
            cs = jnp.sum(m, axis=0, keepdims=True)
        else:
            rowscale = (d <= s).astype(jnp.float32)
            colscale = (d < s).astype(jnp.float32)
            acc = acc + m * rowscale
            cs = jnp.sum(m, axis=0, keepdims=True) * colscale
        colcnt_ref[:, pl.ds(off + s * 128, 128)] += cs
    return acc


def _ghm_body(x_rows_ref, y_rows_ref, x_cols_ref, y_cols_ref, pw_ref,
              wsum_ref, psum_ref, gcols_ref, colcnt_ref, rowcnt_ref):
    i = pl.program_id(0)
    n = x_cols_ref.shape[1]
    ng = pl.num_programs(0)

    @pl.when(i == 0)
    def _init():
        xc = x_cols_ref[...]
        yc = y_cols_ref[...]
        gcols_ref[...] = jnp.abs(jax.nn.sigmoid(xc) - yc)
        colcnt_ref[...] = jnp.zeros_like(colcnt_ref)

    r0 = i * _BR

    for rh in range(_BR // 128):
        x_rh = x_rows_ref[0, rh * 128:(rh + 1) * 128]
        y_rh = y_rows_ref[0, rh * 128:(rh + 1) * 128]
        g_rh = jnp.abs(jax.nn.sigmoid(x_rh) - y_rh)

        gr = r0 // 128 + rh
        c0 = gr // (_BC // 128)
        d = gr - c0 * (_BC // 128)
        off0 = pl.multiple_of(c0 * _BC, _BC)
        acc = _count_chunk(gcols_ref, colcnt_ref, g_rh, off0,
                           jnp.zeros((128, 128), jnp.float32), d)

        def chunk(c, a, g_rh=g_rh):
            off = pl.multiple_of(c * _BC, _BC)
            return _count_chunk(gcols_ref, colcnt_ref, g_rh, off, a)

        acc = jax.lax.fori_loop(c0 + 1, n // _BC, chunk, acc)

        cnt = jnp.sum(acc, axis=1, keepdims=True)
        rowcnt_ref[:, pl.ds(r0 + rh * 128, 128)] = cnt.reshape(1, 128)

    @pl.when(i == ng - 1)
    def _finalize():
        gd = (rowcnt_ref[...] + colcnt_ref[...]) / _DELTA
        beta = n / (gd + _EPS)
        xc = x_cols_ref[...]
        yc = y_cols_ref[...]
        pw = pw_ref[0, 0]
        pe = (pw * yc * jax.nn.softplus(-xc)
              + (1.0 - yc) * jax.nn.softplus(xc))
        wsum_ref[0, 0] = jnp.sum(beta * pe)
        psum_ref[0, 0] = jnp.sum(pe)


def kernel(logits, targets, pos_weight):
    x = logits.reshape(-1).astype(jnp.float32)
    y = targets.reshape(-1).astype(jnp.float32)
    n = x.shape[0]
    g = n // _BR

    x_rows = x.reshape(g, _BR, 1)
    y_rows = y.reshape(g, _BR, 1)
    x_cols = x.reshape(1, n)
    y_cols = y.reshape(1, n)
    pw = jnp.asarray(pos_weight, jnp.float32).reshape(1, 1)

    wsum, psum = pl.pallas_call(
        _ghm_body,
        grid=(g,),
        in_specs=[
            pl.BlockSpec((1, _BR, 1), lambda i: (i, 0, 0)),
            pl.BlockSpec((1, _BR, 1), lambda i: (i, 0, 0)),
            pl.BlockSpec((1, n), lambda i: (0, 0)),
            pl.BlockSpec((1, n), lambda i: (0, 0)),
            pl.BlockSpec(memory_space=pltpu.SMEM),
        ],
        out_specs=[
            pl.BlockSpec((1, 1), lambda i: (0, 0), memory_space=pltpu.SMEM),
            pl.BlockSpec((1, 1), lambda i: (0, 0), memory_space=pltpu.SMEM),
        ],
        out_shape=[
            jax.ShapeDtypeStruct((1, 1), jnp.float32),
            jax.ShapeDtypeStruct((1, 1), jnp.float32),
        ],
        scratch_shapes=[
            pltpu.VMEM((1, n), jnp.float32),
            pltpu.VMEM((1, n), jnp.float32),
            pltpu.VMEM((1, n), jnp.float32),
        ],
        compiler_params=pltpu.CompilerParams(
            dimension_semantics=("arbitrary",),
        ),
        name="ghm_bce",
    )(x_rows, y_rows, x_cols, y_cols, pw)

    inv_n = jnp.float32(1.0 / n)
    return wsum[0, 0] * inv_n, psum[0, 0] * inv_n
